# baseline (speedup 1.0000x reference)
import functools

import jax
import jax.numpy as jnp
from jax import lax
from jax.experimental import pallas as pl
from jax.experimental.pallas import tpu as pltpu
from jax.experimental.pallas import tpu_sc as plsc

_CAT = (119, 9, 12, 12, 10, 6, 6, 2, 2)
_D = 128
_N = 100000
_NC = 2
_NS = 16
_NW = _NC * _NS
_C = 128
_PT = 3200
_NP = _NW * _PT
_CHUNKS = _PT // _C
_GROUPS = _PT // 16

_RA = (_CAT[0] + 1) * (_CAT[1] + 1)
_RB = (_CAT[2] + 1) * (_CAT[3] + 1) * (_CAT[4] + 1)
_RC = (_CAT[5] + 1) * (_CAT[6] + 1) * (_CAT[7] + 1) * (_CAT[8] + 1)
_RT = _RA + _RB + _RC


def _sc_body(xT, tbl, out, xi, ia, ib, ic, ra, rb, rc, shd, sem):
    wid = lax.axis_index("s") * _NC + lax.axis_index("c")
    base = wid * _PT

    @pl.when(lax.axis_index("s") == 0)
    def _stage():
        pltpu.sync_copy(tbl, shd)

    pltpu.sync_copy(xT.at[:, pl.ds(base, _PT)], xi)
    plsc.subcore_barrier()

    def chunk(ci, carry):
        for g in range(8):
            sl = pl.ds(ci * _C + g * 16, 16)
            c = [jnp.minimum(xi[f, sl], _CAT[f]) for f in range(9)]
            a = c[0] * (_CAT[1] + 1) + c[1]
            b = (c[2] * (_CAT[3] + 1) + c[3]) * (_CAT[4] + 1) + c[4] + _RA
            cc = (
                ((c[5] * (_CAT[6] + 1) + c[6]) * (_CAT[7] + 1) + c[7]) * (_CAT[8] + 1)
                + c[8]
                + (_RA + _RB)
            )
            so = pl.ds(g * 16, 16)
            ia[so] = a
            ib[so] = b
            ic[so] = cc
        cpa = pltpu.async_copy(shd.at[ia], ra, sem)
        cpb = pltpu.async_copy(shd.at[ib], rb, sem)
        cpc = pltpu.async_copy(shd.at[ic], rc, sem)
        cpa.wait()
        cpb.wait()
        cpc.wait()

        def srow(j, carry2):
            for k in range(8):
                sl = pl.ds(k * 16, 16)
                ra[j, sl] = ra[j, sl] + rb[j, sl] + rc[j, sl]
            return carry2

        lax.fori_loop(0, _C, srow, 0)
        pltpu.sync_copy(ra, out.at[pl.ds(base + ci * _C, _C)])
        return carry

    lax.fori_loop(0, _CHUNKS, chunk, 0)


_sc_gather = functools.partial(
    pl.kernel,
    mesh=plsc.VectorSubcoreMesh(core_axis_name="c", subcore_axis_name="s"),
    out_type=jax.ShapeDtypeStruct((_NP, _D), jnp.float32),
    scratch_types=[
        pltpu.VMEM((9, _PT), jnp.int32),
        pltpu.VMEM((_C,), jnp.int32),
        pltpu.VMEM((_C,), jnp.int32),
        pltpu.VMEM((_C,), jnp.int32),
        pltpu.VMEM((_C, _D), jnp.float32),
        pltpu.VMEM((_C, _D), jnp.float32),
        pltpu.VMEM((_C, _D), jnp.float32),
        pltpu.VMEM_SHARED((_RT, _D), jnp.float32),
        pltpu.SemaphoreType.DMA,
    ],
)(_sc_body)


_BM = 800


def _tc_body(a_ref, t_ref, w_ref, b_ref, o_ref):
    o_ref[...] = (
        a_ref[...]
        + jnp.dot(t_ref[...], w_ref[...], preferred_element_type=jnp.float32)
        + b_ref[...]
    )


def _tc_call(acc, t, w2, b):
    return pl.pallas_call(
        _tc_body,
        grid=(_N // _BM,),
        in_specs=[
            pl.BlockSpec((_BM, _D), lambda i: (i, 0)),
            pl.BlockSpec((_BM, _D), lambda i: (i, 0)),
            pl.BlockSpec((_D, _D), lambda i: (0, 0)),
            pl.BlockSpec((1, _D), lambda i: (0, 0)),
        ],
        out_specs=pl.BlockSpec((_BM, _D), lambda i: (i, 0)),
        out_shape=jax.ShapeDtypeStruct((_N, _D), jnp.float32),
    )(acc, t, w2, b)


def kernel(x, time_features, emb0, emb1, emb2, emb3, emb4, emb5, emb6, emb7, emb8, W_t, b_t):
    w1 = W_t[:_D] * (1.0 / 3.0)
    w2 = W_t[_D:]
    pa = (emb0[:, None, :] + emb1[None, :, :]).reshape(_RA, _D)
    pb = (emb2[:, None, None, :] + emb3[None, :, None, :] + emb4[None, None, :, :]).reshape(_RB, _D)
    pc = (
        emb5[:, None, None, None, :]
        + emb6[None, :, None, None, :]
        + emb7[None, None, :, None, :]
        + emb8[None, None, None, :, :]
    ).reshape(_RC, _D)
    tbl = jnp.concatenate([pa, pb, pc], axis=0) @ w1
    xT = jnp.pad(x, ((0, _NP - _N), (0, 0))).T
    acc = _sc_gather(xT, tbl)
    return _tc_call(acc, time_features, w2, jnp.reshape(b_t, (1, _D)))


# speedup vs baseline: 1.1411x; 1.1411x over previous
import functools

import jax
import jax.numpy as jnp
from jax import lax
from jax.experimental import pallas as pl
from jax.experimental.pallas import tpu as pltpu
from jax.experimental.pallas import tpu_sc as plsc

_CAT = (119, 9, 12, 12, 10, 6, 6, 2, 2)
_D = 128
_N = 100000
_NC = 2
_NS = 16
_NW = _NC * _NS
_C = 64
_PT = 3200
_NP = _NW * _PT
_CHUNKS = _PT // _C
_GROUPS = _PT // 16

_RA = (_CAT[0] + 1) * (_CAT[1] + 1)
_RB = (_CAT[2] + 1) * (_CAT[3] + 1) * (_CAT[4] + 1)
_RC = (_CAT[5] + 1) * (_CAT[6] + 1) * (_CAT[7] + 1) * (_CAT[8] + 1)
_RT = _RA + _RB + _RC


def _sc_body(xT, tbl, out, xi, ia, ib, ic, ra, rb, rc, shd, sg0, sg1, so0, so1):
    wid = lax.axis_index("s") * _NC + lax.axis_index("c")
    base = wid * _PT
    sg = (sg0, sg1)
    so = (so0, so1)

    @pl.when(lax.axis_index("s") == 0)
    def _stage():
        pltpu.sync_copy(tbl, shd)

    pltpu.sync_copy(xT.at[:, pl.ds(base, _PT)], xi)
    plsc.subcore_barrier()

    def compute_idx(ci, p):
        for g in range(_C // 16):
            sl = pl.ds(ci * _C + g * 16, 16)
            v = [xi[f, sl] for f in range(9)]
            c = [jnp.where((v[f] < 0) | (v[f] >= _CAT[f]), _CAT[f], v[f]) for f in range(9)]
            a = c[0] * (_CAT[1] + 1) + c[1]
            b = (c[2] * (_CAT[3] + 1) + c[3]) * (_CAT[4] + 1) + c[4] + _RA
            cc = (
                ((c[5] * (_CAT[6] + 1) + c[6]) * (_CAT[7] + 1) + c[7]) * (_CAT[8] + 1)
                + c[8]
                + (_RA + _RB)
            )
            sd = pl.ds(g * 16, 16)
            ia[p, sd] = a
            ib[p, sd] = b
            ic[p, sd] = cc

    def fire_gathers(p):
        pltpu.async_copy(shd.at[ia.at[p]], ra.at[p], sg[p])
        pltpu.async_copy(shd.at[ib.at[p]], rb.at[p], sg[p])
        pltpu.async_copy(shd.at[ic.at[p]], rc.at[p], sg[p])

    def drain_gathers(p):
        pltpu.make_async_copy(shd.at[ia.at[p]], ra.at[p], sg[p]).wait()
        pltpu.make_async_copy(shd.at[ib.at[p]], rb.at[p], sg[p]).wait()
        pltpu.make_async_copy(shd.at[ic.at[p]], rc.at[p], sg[p]).wait()

    def out_slice(ci):
        return out.at[pl.ds(base + ci * _C, _C)]

    def process(ci, p):
        drain_gathers(p)

        def srow(j, carry2):
            for k in range(8):
                sl = pl.ds(k * 16, 16)
                ra[p, j, sl] = ra[p, j, sl] + rb[p, j, sl] + rc[p, j, sl]
            return carry2

        lax.fori_loop(0, _C, srow, 0)
        pltpu.async_copy(ra.at[p], out_slice(ci), so[p])

    compute_idx(0, 0)
    fire_gathers(0)

    def pair(k, carry):
        compute_idx(2 * k + 1, 1)

        @pl.when(k >= 1)
        def _wo1():
            pltpu.make_async_copy(ra.at[1], out_slice(2 * k - 1), so[1]).wait()

        fire_gathers(1)
        process(2 * k, 0)

        @pl.when(k < _CHUNKS // 2 - 1)
        def _pf0():
            compute_idx(2 * k + 2, 0)
            pltpu.make_async_copy(ra.at[0], out_slice(2 * k), so[0]).wait()
            fire_gathers(0)

        @pl.when(k == _CHUNKS // 2 - 1)
        def _wo0():
            pltpu.make_async_copy(ra.at[0], out_slice(2 * k), so[0]).wait()

        process(2 * k + 1, 1)
        return carry

    lax.fori_loop(0, _CHUNKS // 2, pair, 0)
    pltpu.make_async_copy(ra.at[1], out_slice(_CHUNKS - 1), so[1]).wait()


_sc_gather = functools.partial(
    pl.kernel,
    mesh=plsc.VectorSubcoreMesh(core_axis_name="c", subcore_axis_name="s"),
    out_type=jax.ShapeDtypeStruct((_NP, _D), jnp.float32),
    scratch_types=[
        pltpu.VMEM((9, _PT), jnp.int32),
        pltpu.VMEM((2, _C), jnp.int32),
        pltpu.VMEM((2, _C), jnp.int32),
        pltpu.VMEM((2, _C), jnp.int32),
        pltpu.VMEM((2, _C, _D), jnp.float32),
        pltpu.VMEM((2, _C, _D), jnp.float32),
        pltpu.VMEM((2, _C, _D), jnp.float32),
        pltpu.VMEM_SHARED((_RT, _D), jnp.float32),
        pltpu.SemaphoreType.DMA,
        pltpu.SemaphoreType.DMA,
        pltpu.SemaphoreType.DMA,
        pltpu.SemaphoreType.DMA,
    ],
)(_sc_body)


_BM = 800


def _tc_body(a_ref, t_ref, w_ref, b_ref, o_ref):
    o_ref[...] = (
        a_ref[...]
        + jnp.dot(t_ref[...], w_ref[...], preferred_element_type=jnp.float32)
        + b_ref[...]
    )


def _tc_call(acc, t, w2, b):
    return pl.pallas_call(
        _tc_body,
        grid=(_N // _BM,),
        in_specs=[
            pl.BlockSpec((_BM, _D), lambda i: (i, 0)),
            pl.BlockSpec((_BM, _D), lambda i: (i, 0)),
            pl.BlockSpec((_D, _D), lambda i: (0, 0)),
            pl.BlockSpec((1, _D), lambda i: (0, 0)),
        ],
        out_specs=pl.BlockSpec((_BM, _D), lambda i: (i, 0)),
        out_shape=jax.ShapeDtypeStruct((_N, _D), jnp.float32),
    )(acc, t, w2, b)


def kernel(x, time_features, emb0, emb1, emb2, emb3, emb4, emb5, emb6, emb7, emb8, W_t, b_t):
    w1 = W_t[:_D] * (1.0 / 3.0)
    w2 = W_t[_D:]
    pa = (emb0[:, None, :] + emb1[None, :, :]).reshape(_RA, _D)
    pb = (emb2[:, None, None, :] + emb3[None, :, None, :] + emb4[None, None, :, :]).reshape(_RB, _D)
    pc = (
        emb5[:, None, None, None, :]
        + emb6[None, :, None, None, :]
        + emb7[None, None, :, None, :]
        + emb8[None, None, None, :, :]
    ).reshape(_RC, _D)
    tbl = jnp.concatenate([pa, pb, pc], axis=0) @ w1
    xT = jnp.pad(x, ((0, _NP - _N), (0, 0))).T
    acc = _sc_gather(xT, tbl)
    return _tc_call(acc, time_features, w2, jnp.reshape(b_t, (1, _D)))
